# baseline (speedup 1.0000x reference)
import functools

import jax
import jax.numpy as jnp
from jax import lax
from jax.experimental import pallas as pl
from jax.experimental.pallas import tpu as pltpu
from jax.experimental.pallas import tpu_sc as plsc

NC = 2
NS = 16
NW = NC * NS

CHUNK = 512


def _sc_gather(idx_flat, table, total, bsz, b_per_w, nchunk, chunk):
    d = table.shape[1]
    half = bsz // 2
    mesh = plsc.VectorSubcoreMesh(core_axis_name="c", subcore_axis_name="s")

    @functools.partial(
        pl.kernel,
        mesh=mesh,
        compiler_params=pltpu.CompilerParams(use_tc_tiling_on_sc=False),
        out_type=jax.ShapeDtypeStruct((total // 2, 2 * d), jnp.float32),
        scratch_types=[
            pltpu.VMEM((chunk,), jnp.int32),
            pltpu.VMEM((chunk,), jnp.int32),
            pltpu.VMEM((chunk, d), jnp.float32),
            pltpu.VMEM((chunk, d), jnp.float32),
            pltpu.SemaphoreType.DMA,
            pltpu.SemaphoreType.DMA,
        ],
    )
    def gather_kernel(idx_hbm, table_hbm, out_hbm, idx0, idx1, buf0, buf1, sem0, sem1):
        wid = lax.axis_index("s") * NC + lax.axis_index("c")
        base = wid * b_per_w
        idxs = (idx0, idx1)
        bufs = (buf0, buf1)
        sems = (sem0, sem1)

        def start(j, s):
            p = base + j * chunk
            li = p // bsz
            pltpu.sync_copy(idx_hbm.at[li, pl.ds(p - li * bsz, chunk)], idxs[s])
            return pltpu.async_copy(table_hbm.at[idxs[s]], bufs[s], sems[s])

        cps = [start(0, 0), None]
        for j in range(nchunk):
            s = j & 1
            if j + 1 < nchunk:
                cps[(j + 1) & 1] = start(j + 1, (j + 1) & 1)
            cps[s].wait()
            p0 = base + j * chunk
            l = p0 // bsz
            off = p0 - l * bsz
            hi = off // half
            row0 = l * half + off - hi * half
            pltpu.sync_copy(
                bufs[s], out_hbm.at[pl.ds(row0, chunk), pl.ds(hi * d, d)]
            )

    return gather_kernel(idx_flat, table)


def _tc_linearize_idx(xt, L, B):
    def body(x_ref, o_ref):
        o_ref[...] = x_ref[...].reshape(L * B // 128, 128)

    return pl.pallas_call(
        body,
        out_shape=jax.ShapeDtypeStruct((L * B // 128, 128), jnp.int32),
    )(xt)


def _tc_linear_t(g128, W, b2, L, B):
    half = B // 2
    bk = 512
    npc = half // bk

    def body(g_ref, w_ref, b_ref, o_ref):
        w = w_ref[...]
        bias = b_ref[...]
        for j in range(npc):
            blk = g_ref[pl.ds(j * bk, bk), :]
            e = lax.dot_general(
                w, blk[:, :64], (((1,), (1,)), ((), ())),
                preferred_element_type=jnp.float32,
            )
            o_ref[0, :, pl.ds(j * bk, bk)] = e + bias
            o = lax.dot_general(
                w, blk[:, 64:], (((1,), (1,)), ((), ())),
                preferred_element_type=jnp.float32,
            )
            o_ref[0, :, pl.ds(half + j * bk, bk)] = o + bias

    return pl.pallas_call(
        body,
        grid=(L,),
        in_specs=[
            pl.BlockSpec((half, 128), lambda i: (i, 0)),
            pl.BlockSpec((64, 64), lambda i: (0, 0)),
            pl.BlockSpec((64, 1), lambda i: (0, 0)),
        ],
        out_specs=pl.BlockSpec((1, 64, B), lambda i: (i, 0, 0)),
        out_shape=jax.ShapeDtypeStruct((L, 64, B), jnp.float32),
    )(g128, W, b2)


def kernel(x, embed_table, W, b):
    bsz, hist = x.shape
    d = embed_table.shape[1]
    total = bsz * hist

    b_per_w = total // NW
    chunk = CHUNK
    nchunk = b_per_w // chunk
    assert b_per_w % chunk == 0 and total % NW == 0 and bsz % 2 == 0
    assert (bsz // 2) % chunk == 0 and bsz % 128 == 0

    idx_lin = _tc_linearize_idx(x.astype(jnp.int32).T, hist, bsz)
    idx2 = idx_lin.reshape(hist, bsz)
    g128 = _sc_gather(idx2, embed_table, total, bsz, b_per_w, nchunk, chunk)

    out_t = _tc_linear_t(g128, W, b.reshape(d, 1), hist, bsz)
    return jnp.transpose(out_t, (2, 0, 1))


# speedup vs baseline: 1.5796x; 1.5796x over previous
import functools

import jax
import jax.numpy as jnp
from jax import lax
from jax.experimental import pallas as pl
from jax.experimental.pallas import tpu as pltpu
from jax.experimental.pallas import tpu_sc as plsc

NC = 2
NS = 16
NW = NC * NS

CHUNK = 512


def _sc_gather(idx_flat, table, total, bsz, b_per_w, nchunk, chunk):
    d = table.shape[1]
    half = bsz // 2
    mesh = plsc.VectorSubcoreMesh(core_axis_name="c", subcore_axis_name="s")

    @functools.partial(
        pl.kernel,
        mesh=mesh,
        compiler_params=pltpu.CompilerParams(use_tc_tiling_on_sc=False),
        out_type=jax.ShapeDtypeStruct((total // 2, 2 * d), jnp.float32),
        scratch_types=[
            pltpu.VMEM((chunk,), jnp.int32),
            pltpu.VMEM((chunk,), jnp.int32),
            pltpu.VMEM((chunk, d), jnp.float32),
            pltpu.VMEM((chunk, d), jnp.float32),
            pltpu.SemaphoreType.DMA,
            pltpu.SemaphoreType.DMA,
        ],
    )
    def gather_kernel(idx_hbm, table_hbm, out_hbm, idx0, idx1, buf0, buf1, sem0, sem1):
        wid = lax.axis_index("s") * NC + lax.axis_index("c")
        base = wid * b_per_w
        idxs = (idx0, idx1)
        bufs = (buf0, buf1)
        sems = (sem0, sem1)

        def start(j, s):
            p = base + j * chunk
            li = p // bsz
            pltpu.sync_copy(idx_hbm.at[li, pl.ds(p - li * bsz, chunk)], idxs[s])
            return pltpu.async_copy(table_hbm.at[idxs[s]], bufs[s], sems[s])

        cps = [start(0, 0), None]
        for j in range(nchunk):
            s = j & 1
            if j + 1 < nchunk:
                cps[(j + 1) & 1] = start(j + 1, (j + 1) & 1)
            cps[s].wait()
            p0 = base + j * chunk
            l = p0 // bsz
            off = p0 - l * bsz
            hi = off // half
            row0 = l * half + off - hi * half
            pltpu.sync_copy(
                bufs[s], out_hbm.at[pl.ds(row0, chunk), pl.ds(hi * d, d)]
            )

    return gather_kernel(idx_flat, table)


def _tc_linearize_idx(xt, L, B, blk, h):
    log2h = h.bit_length() - 1

    def body(x_ref, o_ref):
        v = x_ref[...].reshape(L * B // 128, 128)
        o_ref[...] = (v & ~(blk - 1)) | ((v & (h - 1)) << 1) | ((v >> log2h) & 1)

    return pl.pallas_call(
        body,
        out_shape=jax.ShapeDtypeStruct((L * B // 128, 128), jnp.int32),
    )(xt)


def _tc_build_p(tableT, W, b1, V, d, blk):
    h = blk // 2
    nblk = -(-V // blk)

    def body(t_ref, w_ref, b_ref, o_ref):
        w = w_ref[...]
        bias = b_ref[...]
        t = t_ref[...]
        e1 = lax.dot_general(
            t[:, :h], w, (((0,), (1,)), ((), ())),
            preferred_element_type=jnp.float32,
        )
        o_ref[:, :d] = e1 + bias
        e2 = lax.dot_general(
            t[:, h:], w, (((0,), (1,)), ((), ())),
            preferred_element_type=jnp.float32,
        )
        o_ref[:, d:] = e2 + bias

    return pl.pallas_call(
        body,
        grid=(nblk,),
        in_specs=[
            pl.BlockSpec((d, blk), lambda i: (0, i)),
            pl.BlockSpec((d, d), lambda i: (0, 0)),
            pl.BlockSpec((1, d), lambda i: (0, 0)),
        ],
        out_specs=pl.BlockSpec((h, 2 * d), lambda i: (i, 0)),
        out_shape=jax.ShapeDtypeStruct((nblk * h, 2 * d), jnp.float32),
    )(tableT, W, b1)


def _tc_linear_t(g128, W, b2, L, B):
    half = B // 2
    bk = 512
    npc = half // bk

    def body(g_ref, w_ref, b_ref, o_ref):
        w = w_ref[...]
        bias = b_ref[...]
        for j in range(npc):
            blk = g_ref[pl.ds(j * bk, bk), :]
            e = lax.dot_general(
                w, blk[:, :64], (((1,), (1,)), ((), ())),
                preferred_element_type=jnp.float32,
            )
            o_ref[0, :, pl.ds(j * bk, bk)] = e + bias
            o = lax.dot_general(
                w, blk[:, 64:], (((1,), (1,)), ((), ())),
                preferred_element_type=jnp.float32,
            )
            o_ref[0, :, pl.ds(half + j * bk, bk)] = o + bias

    return pl.pallas_call(
        body,
        grid=(L,),
        in_specs=[
            pl.BlockSpec((half, 128), lambda i: (i, 0)),
            pl.BlockSpec((64, 64), lambda i: (0, 0)),
            pl.BlockSpec((64, 1), lambda i: (0, 0)),
        ],
        out_specs=pl.BlockSpec((1, 64, B), lambda i: (i, 0, 0)),
        out_shape=jax.ShapeDtypeStruct((L, 64, B), jnp.float32),
    )(g128, W, b2)


def kernel(x, embed_table, W, b):
    bsz, hist = x.shape
    d = embed_table.shape[1]
    total = bsz * hist

    b_per_w = total // NW
    chunk = CHUNK
    nchunk = b_per_w // chunk
    assert b_per_w % chunk == 0 and total % NW == 0 and bsz % 2 == 0
    assert (bsz // 2) % chunk == 0 and bsz % 128 == 0

    V = embed_table.shape[0]
    blk = 4096
    h2 = blk // 2
    nblk = -(-V // blk)
    assert (h2 & (h2 - 1)) == 0

    p = _tc_build_p(embed_table.T, W, b.reshape(1, d), V, d, blk)
    p2 = p.reshape(nblk * blk, d)

    idx_lin = _tc_linearize_idx(x.astype(jnp.int32).T, hist, bsz, blk, h2)
    idx2 = idx_lin.reshape(hist, bsz)
    g128 = _sc_gather(idx2, p2, total, bsz, b_per_w, nchunk, chunk)

    eye = jnp.eye(d, dtype=jnp.float32)
    out_t = _tc_linear_t(g128, eye, jnp.zeros((d, 1), jnp.float32), hist, bsz)
    return jnp.transpose(out_t, (2, 0, 1))


# speedup vs baseline: 2.0000x; 1.2662x over previous
import functools

import jax
import jax.numpy as jnp
from jax import lax
from jax.experimental import pallas as pl
from jax.experimental.pallas import tpu as pltpu
from jax.experimental.pallas import tpu_sc as plsc

NC = 2
NS = 16
NW = NC * NS

CHUNK = 512


def _sc_gather(idx_flat, table, total, bsz, b_per_w, nchunk, chunk):
    d = table.shape[1]
    half = bsz // 2
    mesh = plsc.VectorSubcoreMesh(core_axis_name="c", subcore_axis_name="s")

    @functools.partial(
        pl.kernel,
        mesh=mesh,
        compiler_params=pltpu.CompilerParams(use_tc_tiling_on_sc=False),
        out_type=jax.ShapeDtypeStruct((total // 2, 2 * d), jnp.float32),
        scratch_types=[
            pltpu.VMEM((chunk,), jnp.int32),
            pltpu.VMEM((chunk,), jnp.int32),
            pltpu.VMEM((chunk, d), jnp.float32),
            pltpu.VMEM((chunk, d), jnp.float32),
            pltpu.SemaphoreType.DMA,
            pltpu.SemaphoreType.DMA,
        ],
    )
    def gather_kernel(idx_hbm, table_hbm, out_hbm, idx0, idx1, buf0, buf1, sem0, sem1):
        wid = lax.axis_index("s") * NC + lax.axis_index("c")
        base = wid * b_per_w
        idxs = (idx0, idx1)
        bufs = (buf0, buf1)
        sems = (sem0, sem1)

        def start(j, s):
            p = base + j * chunk
            li = p // bsz
            pltpu.sync_copy(idx_hbm.at[li, pl.ds(p - li * bsz, chunk)], idxs[s])
            return pltpu.async_copy(table_hbm.at[idxs[s]], bufs[s], sems[s])

        cps = [start(0, 0), None]
        for j in range(nchunk):
            s = j & 1
            if j + 1 < nchunk:
                cps[(j + 1) & 1] = start(j + 1, (j + 1) & 1)
            cps[s].wait()
            p0 = base + j * chunk
            l = p0 // bsz
            off = p0 - l * bsz
            hi = off // half
            row0 = l * half + off - hi * half
            pltpu.sync_copy(
                bufs[s], out_hbm.at[pl.ds(row0, chunk), pl.ds(hi * d, d)]
            )

    return gather_kernel(idx_flat, table)


def _tc_linearize_idx(xt, L, B, blk, h):
    log2h = h.bit_length() - 1

    def body(x_ref, o_ref):
        v = x_ref[...].reshape(L * B // 128, 128)
        o_ref[...] = (v & ~(blk - 1)) | ((v & (h - 1)) << 1) | ((v >> log2h) & 1)

    return pl.pallas_call(
        body,
        out_shape=jax.ShapeDtypeStruct((L * B // 128, 128), jnp.int32),
    )(xt)


def _tc_build_p(tableT, W, b1, V, d, blk):
    h = blk // 2
    nblk = -(-V // blk)

    def body(t_ref, w_ref, b_ref, o_ref):
        w = w_ref[...].astype(jnp.bfloat16)
        bias = b_ref[...]
        t = t_ref[...].astype(jnp.bfloat16)
        e1 = lax.dot_general(
            t[:, :h], w, (((0,), (1,)), ((), ())),
            preferred_element_type=jnp.float32,
        )
        o_ref[:, :d] = e1 + bias
        e2 = lax.dot_general(
            t[:, h:], w, (((0,), (1,)), ((), ())),
            preferred_element_type=jnp.float32,
        )
        o_ref[:, d:] = e2 + bias

    return pl.pallas_call(
        body,
        grid=(nblk,),
        in_specs=[
            pl.BlockSpec((d, blk), lambda i: (0, i)),
            pl.BlockSpec((d, d), lambda i: (0, 0)),
            pl.BlockSpec((1, d), lambda i: (0, 0)),
        ],
        out_specs=pl.BlockSpec((h, 2 * d), lambda i: (i, 0)),
        out_shape=jax.ShapeDtypeStruct((nblk * h, 2 * d), jnp.float32),
    )(tableT, W, b1)


def _tc_linear_t(g128, W, b2, L, B):
    half = B // 2
    bk = 512
    npc = half // bk

    def body(g_ref, w_ref, b_ref, o_ref):
        w = w_ref[...]
        bias = b_ref[...]
        for j in range(npc):
            blk = g_ref[pl.ds(j * bk, bk), :]
            e = lax.dot_general(
                w, blk[:, :64], (((1,), (1,)), ((), ())),
                preferred_element_type=jnp.float32,
            )
            o_ref[0, :, pl.ds(j * bk, bk)] = e + bias
            o = lax.dot_general(
                w, blk[:, 64:], (((1,), (1,)), ((), ())),
                preferred_element_type=jnp.float32,
            )
            o_ref[0, :, pl.ds(half + j * bk, bk)] = o + bias

    return pl.pallas_call(
        body,
        grid=(L,),
        in_specs=[
            pl.BlockSpec((half, 128), lambda i: (i, 0)),
            pl.BlockSpec((64, 64), lambda i: (0, 0)),
            pl.BlockSpec((64, 1), lambda i: (0, 0)),
        ],
        out_specs=pl.BlockSpec((1, 64, B), lambda i: (i, 0, 0)),
        out_shape=jax.ShapeDtypeStruct((L, 64, B), jnp.float32),
    )(g128, W, b2)


def kernel(x, embed_table, W, b):
    bsz, hist = x.shape
    d = embed_table.shape[1]
    total = bsz * hist

    b_per_w = total // NW
    chunk = CHUNK
    nchunk = b_per_w // chunk
    assert b_per_w % chunk == 0 and total % NW == 0 and bsz % 2 == 0
    assert (bsz // 2) % chunk == 0 and bsz % 128 == 0

    V = embed_table.shape[0]
    blk = 8192
    h2 = blk // 2
    nblk = -(-V // blk)
    assert (h2 & (h2 - 1)) == 0

    p = _tc_build_p(embed_table.T, W, b.reshape(1, d), V, d, blk)
    p2 = p.reshape(nblk * blk, d)

    idx_lin = _tc_linearize_idx(x.astype(jnp.int32).T, hist, bsz, blk, h2)
    idx2 = idx_lin.reshape(hist, bsz)
    g128 = _sc_gather(idx2, p2, total, bsz, b_per_w, nchunk, chunk)

    eye = jnp.eye(d, dtype=jnp.float32)
    out_t = _tc_linear_t(g128, eye, jnp.zeros((d, 1), jnp.float32), hist, bsz)
    return jnp.transpose(out_t, (2, 0, 1))
